# baseline (speedup 1.0000x reference)
import jax
import jax.numpy as jnp
from jax import lax
from jax.experimental import pallas as pl
from jax.experimental.pallas import tpu as pltpu
from jax.experimental.pallas import tpu_sc as plsc

B, CIN, COUT, M = 4, 3, 64, 16
ROW = 16
CH = 16
WROW = M * COUT * CIN
NW = 32
NT_MAX = 20


def _sc_conv(xt_hbm, nbr_hbm, w_hbm, bias_hbm, out_hbm,
             idx_all, g_v, w_v, out_v, bias_v,
             sem_pre, sem_g0, sem_g1, sem_w0, sem_w1, sem_o0, sem_o1):
    npts = w_hbm.shape[0]
    nchunks = npts // CH
    wid = lax.axis_index("s") * 2 + lax.axis_index("c")
    nt = (nchunks - wid + NW - 1) // NW
    sem_g = [sem_g0, sem_g1]
    sem_w = [sem_w0, sem_w1]
    sem_o = [sem_o0, sem_o1]

    pltpu.sync_copy(bias_hbm, bias_v)
    bias_regs = [bias_v[pl.ds(og * 16, 16)] for og in range(4)]
    oi3 = lax.iota(jnp.int32, 16) * 3
    lane_consts = [jnp.full((16,), c, jnp.int32) for c in range(B * CIN)]

    for t in range(NT_MAX):
        @pl.when(t < nt)
        def _():
            for j in range(2):
                pltpu.async_copy(
                    nbr_hbm.at[pl.ds((wid + t * NW) * 256 + j * 128, 128)],
                    idx_all.at[t, j], sem_pre)
    for t in range(NT_MAX):
        @pl.when(t < nt)
        def _():
            for j in range(2):
                pltpu.make_async_copy(
                    nbr_hbm.at[pl.ds(0, 128)], idx_all.at[t, j],
                    sem_pre).wait()

    idx2 = idx_all

    def issue_chunk(t, sl):
        c = wid + t * NW
        for j in range(2):
            pltpu.async_copy(xt_hbm.at[idx2.at[t, j]],
                             g_v.at[sl, pl.ds(j * 128, 128)], sem_g[sl])
        pltpu.async_copy(w_hbm.at[pl.ds(c * CH, CH)], w_v.at[sl],
                         sem_w[sl])

    def wait_chunk(t, sl):
        for j in range(2):
            pltpu.make_async_copy(
                xt_hbm.at[idx2.at[t, j]],
                g_v.at[sl, pl.ds(j * 128, 128)], sem_g[sl]).wait()
        pltpu.make_async_copy(w_hbm.at[pl.ds(0, CH)], w_v.at[sl],
                              sem_w[sl]).wait()

    def compute_chunk(t, sl):
        @plsc.parallel_loop(0, CH, unroll=2)
        def point_body(q):
            acc = [[jnp.zeros((16,), jnp.float32) for _ in range(4)]
                   for _ in range(B)]
            for m in range(M):
                gv = g_v[sl, q * 16 + m]
                for i in range(CIN):
                    wvecs = [plsc.load_gather(
                        w_v.at[sl],
                        [jnp.full((16,), q, jnp.int32),
                         oi3 + (m * 192 + og * 48 + i)])
                        for og in range(4)]
                    for b in range(B):
                        s = gv.at[lane_consts[b * CIN + i]].get(
                            mode="promise_in_bounds")
                        for og in range(4):
                            acc[b][og] = acc[b][og] + wvecs[og] * s
            for b in range(B):
                for og in range(4):
                    v = acc[b][og] + bias_regs[og]
                    r = jnp.where(v > 0, v, jnp.exp(v) - 1.0)
                    out_v[sl, b, q, pl.ds(og * 16, 16)] = r

    def drain_out(sl):
        for b in range(B):
            pltpu.make_async_copy(
                out_v.at[sl, b], out_hbm.at[b, pl.ds(0, CH)],
                sem_o[sl]).wait()

    def issue_out(t, sl):
        c = wid + t * NW
        for b in range(B):
            pltpu.async_copy(out_v.at[sl, b],
                             out_hbm.at[b, pl.ds(c * CH, CH)], sem_o[sl])

    issue_chunk(0, 0)

    def pair_body(tt, _):
        for s in range(2):
            t = tt * 2 + s

            @pl.when(t + 1 < nt)
            def _():
                issue_chunk(t + 1, 1 - s)

            @pl.when(t < nt)
            def _():
                wait_chunk(t, s)

                @pl.when(t >= 2)
                def _():
                    drain_out(s)

                compute_chunk(t, s)
                issue_out(t, s)
        return 0

    lax.fori_loop(0, (NT_MAX + 1) // 2, pair_body, 0)
    drain_out(0)
    drain_out(1)


def kernel(x_batch, neighbor_id_lstlst, weights, bias):
    _, N, _ = x_batch.shape
    xt = jnp.pad(x_batch.transpose(1, 0, 2).reshape(N, B * CIN),
                 ((0, 0), (0, ROW - B * CIN)))
    nbr_flat = neighbor_id_lstlst.reshape(N * M).astype(jnp.int32)
    w_flat = weights.reshape(N, WROW)
    mesh = plsc.VectorSubcoreMesh(core_axis_name="c", subcore_axis_name="s")
    f = pl.kernel(
        _sc_conv,
        mesh=mesh,
        compiler_params=pltpu.CompilerParams(use_tc_tiling_on_sc=False,
                                             needs_layout_passes=False),
        out_type=jax.ShapeDtypeStruct((B, N, COUT), jnp.float32),
        scratch_types=[
            pltpu.VMEM((NT_MAX, 2, 128), jnp.int32),
            pltpu.VMEM((2, 256, ROW), jnp.float32),
            pltpu.VMEM((2, CH, WROW), jnp.float32),
            pltpu.VMEM((2, B, CH, COUT), jnp.float32),
            pltpu.VMEM((COUT,), jnp.float32),
            pltpu.SemaphoreType.DMA,
            pltpu.SemaphoreType.DMA,
            pltpu.SemaphoreType.DMA,
            pltpu.SemaphoreType.DMA,
            pltpu.SemaphoreType.DMA,
            pltpu.SemaphoreType.DMA,
            pltpu.SemaphoreType.DMA,
        ],
    )
    return f(xt, nbr_flat, w_flat, bias)


# speedup vs baseline: 1.3078x; 1.3078x over previous
import jax
import jax.numpy as jnp
from jax import lax
from jax.experimental import pallas as pl
from jax.experimental.pallas import tpu as pltpu
from jax.experimental.pallas import tpu_sc as plsc

B, CIN, COUT, M = 4, 3, 64, 16
ROW = 16
CH = 16
WROW = M * COUT * CIN
NW = 32
NT_MAX = 20


def _sc_conv(xt_hbm, nbr_hbm, w_hbm, bias_hbm, out_hbm,
             idx_all, g_v, w_v, out_v, bias_v,
             sem_pre, sem_g0, sem_g1, sem_w0, sem_w1, sem_o0, sem_o1):
    npts = w_hbm.shape[0]
    nchunks = npts // CH
    wid = lax.axis_index("s") * 2 + lax.axis_index("c")
    nt = (nchunks - wid + NW - 1) // NW
    sem_g = [sem_g0, sem_g1]
    sem_w = [sem_w0, sem_w1]
    sem_o = [sem_o0, sem_o1]

    pltpu.sync_copy(bias_hbm, bias_v)
    bias_regs = [bias_v[pl.ds(og * 16, 16)] for og in range(4)]
    oi3 = lax.iota(jnp.int32, 16) * 3
    lane_consts = [jnp.full((16,), c, jnp.int32) for c in range(B * CIN)]

    for t in range(NT_MAX):
        @pl.when(t < nt)
        def _():
            for j in range(2):
                pltpu.async_copy(
                    nbr_hbm.at[pl.ds((wid + t * NW) * 256 + j * 128, 128)],
                    idx_all.at[t, j], sem_pre)
    for t in range(NT_MAX):
        @pl.when(t < nt)
        def _():
            for j in range(2):
                pltpu.make_async_copy(
                    nbr_hbm.at[pl.ds(0, 128)], idx_all.at[t, j],
                    sem_pre).wait()

    idx2 = idx_all

    def issue_chunk(t, sl):
        c = wid + t * NW
        for j in range(2):
            pltpu.async_copy(xt_hbm.at[idx2.at[t, j]],
                             g_v.at[sl, pl.ds(j * 128, 128)], sem_g[sl])
        pltpu.async_copy(w_hbm.at[pl.ds(c * CH, CH)], w_v.at[sl],
                         sem_w[sl])

    def wait_chunk(t, sl):
        for j in range(2):
            pltpu.make_async_copy(
                xt_hbm.at[idx2.at[t, j]],
                g_v.at[sl, pl.ds(j * 128, 128)], sem_g[sl]).wait()
        pltpu.make_async_copy(w_hbm.at[pl.ds(0, CH)], w_v.at[sl],
                              sem_w[sl]).wait()

    def compute_chunk(t, sl):
        @plsc.parallel_loop(0, CH, unroll=1)
        def point_body(q):
            acc = [[jnp.zeros((16,), jnp.float32) for _ in range(4)]
                   for _ in range(B)]
            for m in range(M):
                gv = g_v[sl, q * 16 + m]
                for i in range(CIN):
                    wvecs = [plsc.load_gather(
                        w_v.at[sl],
                        [jnp.full((16,), q, jnp.int32),
                         oi3 + (m * 192 + og * 48 + i)])
                        for og in range(4)]
                    for b in range(B):
                        s = gv.at[lane_consts[b * CIN + i]].get(
                            mode="promise_in_bounds")
                        for og in range(4):
                            acc[b][og] = acc[b][og] + wvecs[og] * s
            for b in range(B):
                for og in range(4):
                    v = acc[b][og] + bias_regs[og]
                    r = jnp.where(v > 0, v, jnp.exp(v) - 1.0)
                    out_v[sl, b, q, pl.ds(og * 16, 16)] = r

    def drain_out(sl):
        for b in range(B):
            pltpu.make_async_copy(
                out_v.at[sl, b], out_hbm.at[b, pl.ds(0, CH)],
                sem_o[sl]).wait()

    def issue_out(t, sl):
        c = wid + t * NW
        for b in range(B):
            pltpu.async_copy(out_v.at[sl, b],
                             out_hbm.at[b, pl.ds(c * CH, CH)], sem_o[sl])

    issue_chunk(0, 0)

    def pair_body(tt, _):
        for s in range(2):
            t = tt * 2 + s

            @pl.when(t + 1 < nt)
            def _():
                issue_chunk(t + 1, 1 - s)

            @pl.when(t < nt)
            def _():
                wait_chunk(t, s)

                @pl.when(t >= 2)
                def _():
                    drain_out(s)

                compute_chunk(t, s)
                issue_out(t, s)
        return 0

    lax.fori_loop(0, (NT_MAX + 1) // 2, pair_body, 0)
    drain_out(0)
    drain_out(1)


def kernel(x_batch, neighbor_id_lstlst, weights, bias):
    _, N, _ = x_batch.shape
    xt = jnp.pad(x_batch.transpose(1, 0, 2).reshape(N, B * CIN),
                 ((0, 0), (0, ROW - B * CIN)))
    nbr_flat = neighbor_id_lstlst.reshape(N * M).astype(jnp.int32)
    w_flat = weights.reshape(N, WROW)
    mesh = plsc.VectorSubcoreMesh(core_axis_name="c", subcore_axis_name="s")
    f = pl.kernel(
        _sc_conv,
        mesh=mesh,
        compiler_params=pltpu.CompilerParams(use_tc_tiling_on_sc=False,
                                             needs_layout_passes=False),
        out_type=jax.ShapeDtypeStruct((B, N, COUT), jnp.float32),
        scratch_types=[
            pltpu.VMEM((NT_MAX, 2, 128), jnp.int32),
            pltpu.VMEM((2, 256, ROW), jnp.float32),
            pltpu.VMEM((2, CH, WROW), jnp.float32),
            pltpu.VMEM((2, B, CH, COUT), jnp.float32),
            pltpu.VMEM((COUT,), jnp.float32),
            pltpu.SemaphoreType.DMA,
            pltpu.SemaphoreType.DMA,
            pltpu.SemaphoreType.DMA,
            pltpu.SemaphoreType.DMA,
            pltpu.SemaphoreType.DMA,
            pltpu.SemaphoreType.DMA,
            pltpu.SemaphoreType.DMA,
        ],
    )
    return f(xt, nbr_flat, w_flat, bias)
